# baseline (speedup 1.0000x reference)
import functools

import jax
import jax.numpy as jnp
from jax import lax
from jax.experimental import pallas as pl
from jax.experimental.pallas import tpu as pltpu
from jax.experimental.pallas import tpu_sc as plsc

N_NODES = 1000000
H = 64
NUM_RELS = 1000
B = 16384
NEG = 20
BN = B * NEG

NC = 2
NS = 16
L = 16
NW = NC * NS

POS_PER_W = B // NW
NEG_PER_W = BN // NW
CHUNK = 160
NCHUNK = NEG_PER_W // CHUNK
NPAIR = NCHUNK // 2
PCHUNK = 128

NB_TC = 16384
GRID_TC = (N_NODES + NB_TC - 1) // NB_TC
HALF_TC = NB_TC // 2
WIDE_ROWS = GRID_TC * HALF_TC


def _pack_body(in_ref, out_ref):
    blk = in_ref[...]
    eye = jnp.eye(H, dtype=jnp.float32)
    t = lax.dot_general(blk, eye, (((0,), (0,)), ((), ())),
                        preferred_element_type=jnp.float32)
    out_ref[:, pl.ds(0, H)] = t[0:HALF_TC]
    out_ref[:, pl.ds(H, H)] = t[HALF_TC:NB_TC]


def _pack(embT):
    return pl.pallas_call(
        _pack_body,
        grid=(GRID_TC,),
        in_specs=[pl.BlockSpec((H, NB_TC), lambda c: (0, c))],
        out_specs=pl.BlockSpec((HALF_TC, 2 * H), lambda c: (c, 0)),
        out_shape=jax.ShapeDtypeStruct((WIDE_ROWS, 2 * H), jnp.float32),
    )(embT)


def _xform(src, off, gidx, par, n):
    for v in range(n // L):
        g = src[pl.ds(off + L * v, L)]
        par[pl.ds(L * v, L)] = (g >> 13) & 1
        gidx[pl.ds(L * v, L)] = ((g >> 14) << 13) | (g & 8191)


def _score_rows(heads, tails, parh, part, w_for_row, scores, psum, t,
                score_off):
    lane = jnp.arange(L, dtype=jnp.int32)
    rbase = t * 16
    pvh = parh[pl.ds(rbase, L)]
    pvt = part[pl.ds(rbase, L)]
    for p in range(16):
        rl = rbase + p
        bh = H * pvh[p]
        bt = H * pvt[p]
        acc = jnp.zeros((L,), jnp.float32)
        for k in range(H // L):
            hk = heads[rl, pl.ds(bh + 16 * k, 16)]
            tk = tails[rl, pl.ds(bt + 16 * k, 16)]
            wk = w_for_row(rl, k, lane)
            acc = acc + hk * wk * tk
        psum[p, pl.ds(0, L)] = acc
    svec = jnp.zeros((L,), jnp.float32)
    for j in range(L):
        svec = svec + plsc.load_gather(
            psum, [lane, jnp.full((L,), j, jnp.int32)])
    scores[pl.ds(score_off + rbase, 16)] = svec
    return 0


def _sc_body(emb, w_pad, src_pos, dst_pos, etype, src_neg, dst_neg,
             pos_out, neg_out,
             pidx, didx, eidx, hidx0, hidx1, tidx0, tidx1,
             parh0, parh1, part0, part1,
             wrows, heads0, heads1, tails0, tails1,
             pos_sc, neg_sc, psum,
             sem_w, sem_g0, sem_g1, sem_i0, sem_i1, sem_s):
    wid = lax.axis_index("s") * NC + lax.axis_index("c")
    pos0 = wid * POS_PER_W
    neg_base = wid * NEG_PER_W

    pltpu.sync_copy(etype.at[pl.ds(pos0, POS_PER_W)], eidx)
    pltpu.sync_copy(src_pos.at[pl.ds(pos0, POS_PER_W)], pidx)
    pltpu.sync_copy(dst_pos.at[pl.ds(pos0, POS_PER_W)], didx)
    for r in range(POS_PER_W // PCHUNK):
        pltpu.async_copy(w_pad.at[eidx.at[pl.ds(r * PCHUNK, PCHUNK)]],
                         heads0.at[pl.ds(0, PCHUNK)], sem_w)
        pltpu.make_async_copy(w_pad.at[eidx.at[pl.ds(r * PCHUNK, PCHUNK)]],
                              heads0.at[pl.ds(0, PCHUNK)], sem_w).wait()

        def wcomp(i, carry, _r=r):
            b = _r * PCHUNK + i
            for k in range(H // L):
                wrows[b >> 1, pl.ds(H * (b & 1) + 16 * k, 16)] = (
                    heads0[i, pl.ds(16 * k, 16)])
            return carry

        lax.fori_loop(0, PCHUNK, wcomp, 0)

    for r in range(POS_PER_W // PCHUNK):
        _xform(pidx, r * PCHUNK, hidx0, parh0, PCHUNK)
        _xform(didx, r * PCHUNK, tidx0, part0, PCHUNK)
        hsl = hidx0.at[pl.ds(0, PCHUNK)]
        tsl = tidx0.at[pl.ds(0, PCHUNK)]
        pltpu.async_copy(emb.at[hsl], heads0.at[pl.ds(0, PCHUNK)], sem_g0)
        pltpu.async_copy(emb.at[tsl], tails0.at[pl.ds(0, PCHUNK)], sem_g0)
        pltpu.make_async_copy(emb.at[hsl], heads0.at[pl.ds(0, PCHUNK)],
                              sem_g0).wait()
        pltpu.make_async_copy(emb.at[tsl], tails0.at[pl.ds(0, PCHUNK)],
                              sem_g0).wait()

        def w_pos(rl, k, lane, _r=r):
            b = _r * PCHUNK + rl
            f = H * b + 16 * k + lane
            return plsc.load_gather(wrows, [f >> 7, f & 127])

        def pos_tile(t, carry, _r=r):
            return _score_rows(heads0, tails0, parh0, part0, w_pos,
                               pos_sc, psum, t, _r * PCHUNK)

        lax.fori_loop(0, PCHUNK // 16, pos_tile, 0)
    pltpu.sync_copy(pos_sc, pos_out.at[pl.ds(pos0, POS_PER_W)])

    def fire_idx(c, hb, tb, sem):
        off = neg_base + c * CHUNK
        pltpu.async_copy(src_neg.at[pl.ds(off, CHUNK)], hb, sem)
        pltpu.async_copy(dst_neg.at[pl.ds(off, CHUNK)], tb, sem)

    def wait_idx(hb, tb, sem):
        pltpu.make_async_copy(src_neg.at[pl.ds(0, CHUNK)], hb, sem).wait()
        pltpu.make_async_copy(dst_neg.at[pl.ds(0, CHUNK)], tb, sem).wait()

    def xform_fire(hb, tb, ph, pt, hd, td, sem):
        _xform(hb, 0, hb, ph, CHUNK)
        _xform(tb, 0, tb, pt, CHUNK)
        pltpu.async_copy(emb.at[hb], hd, sem)
        pltpu.async_copy(emb.at[tb], td, sem)

    def wait_g(hb, tb, hd, td, sem):
        pltpu.make_async_copy(emb.at[hb], hd, sem).wait()
        pltpu.make_async_copy(emb.at[tb], td, sem).wait()

    def compute(c, hd, td, ph, pt, soff):
        def w_neg(rl, k, lane):
            fvec = lax.div(64 * (c * CHUNK + rl) + 16 * k + lane,
                           jnp.int32(20))
            return plsc.load_gather(wrows, [fvec >> 7, fvec & 127])

        def neg_tile(t, carry):
            return _score_rows(hd, td, ph, pt, w_neg, neg_sc, psum, t,
                               soff)

        lax.fori_loop(0, CHUNK // 16, neg_tile, 0)

    fire_idx(0, hidx0, tidx0, sem_i0)
    fire_idx(1, hidx1, tidx1, sem_i1)
    wait_idx(hidx0, tidx0, sem_i0)
    xform_fire(hidx0, tidx0, parh0, part0, heads0, tails0, sem_g0)
    wait_idx(hidx1, tidx1, sem_i1)

    def sc_out(c0):
        return neg_out.at[pl.ds(neg_base + c0 * CHUNK, 2 * CHUNK)]

    def pair(i, carry):
        c0 = 2 * i
        c1 = c0 + 1
        not_last = i < NPAIR - 1
        xform_fire(hidx1, tidx1, parh1, part1, heads1, tails1, sem_g1)
        wait_g(hidx0, tidx0, heads0, tails0, sem_g0)

        @pl.when(i > 0)
        def _():
            pltpu.make_async_copy(sc_out(c0 - 2), neg_sc, sem_s).wait()

        @pl.when(not_last)
        def _():
            fire_idx(c0 + 2, hidx0, tidx0, sem_i0)

        compute(c0, heads0, tails0, parh0, part0, 0)

        @pl.when(not_last)
        def _():
            wait_idx(hidx0, tidx0, sem_i0)
            xform_fire(hidx0, tidx0, parh0, part0, heads0, tails0, sem_g0)

        wait_g(hidx1, tidx1, heads1, tails1, sem_g1)

        @pl.when(not_last)
        def _():
            fire_idx(c1 + 2, hidx1, tidx1, sem_i1)

        compute(c1, heads1, tails1, parh1, part1, CHUNK)

        @pl.when(not_last)
        def _():
            wait_idx(hidx1, tidx1, sem_i1)

        pltpu.async_copy(neg_sc, sc_out(c0), sem_s)
        return carry

    lax.fori_loop(0, NPAIR, pair, 0)
    pltpu.make_async_copy(sc_out(NCHUNK - 2), neg_sc, sem_s).wait()


_sc_score = functools.partial(
    pl.kernel,
    out_type=(jax.ShapeDtypeStruct((B,), jnp.float32),
              jax.ShapeDtypeStruct((BN,), jnp.float32)),
    mesh=plsc.VectorSubcoreMesh(core_axis_name="c", subcore_axis_name="s"),
    compiler_params=pltpu.CompilerParams(needs_layout_passes=False,
                                         use_tc_tiling_on_sc=True),
    scratch_types=(
        pltpu.VMEM((POS_PER_W,), jnp.int32),
        pltpu.VMEM((POS_PER_W,), jnp.int32),
        pltpu.VMEM((POS_PER_W,), jnp.int32),
        pltpu.VMEM((CHUNK,), jnp.int32),
        pltpu.VMEM((CHUNK,), jnp.int32),
        pltpu.VMEM((CHUNK,), jnp.int32),
        pltpu.VMEM((CHUNK,), jnp.int32),
        pltpu.VMEM((CHUNK,), jnp.int32),
        pltpu.VMEM((CHUNK,), jnp.int32),
        pltpu.VMEM((CHUNK,), jnp.int32),
        pltpu.VMEM((CHUNK,), jnp.int32),
        pltpu.VMEM((POS_PER_W // 2, 2 * H), jnp.float32),
        pltpu.VMEM((CHUNK, 2 * H), jnp.float32),
        pltpu.VMEM((CHUNK, 2 * H), jnp.float32),
        pltpu.VMEM((CHUNK, 2 * H), jnp.float32),
        pltpu.VMEM((CHUNK, 2 * H), jnp.float32),
        pltpu.VMEM((POS_PER_W,), jnp.float32),
        pltpu.VMEM((2 * CHUNK,), jnp.float32),
        pltpu.VMEM((L, 128), jnp.float32),
        pltpu.SemaphoreType.DMA,
        pltpu.SemaphoreType.DMA,
        pltpu.SemaphoreType.DMA,
        pltpu.SemaphoreType.DMA,
        pltpu.SemaphoreType.DMA,
        pltpu.SemaphoreType.DMA,
    ),
)(_sc_body)


def _loss_body(pos_ref, neg_ref, out_ref):
    def softplus(z):
        return jnp.maximum(z, 0.0) + jnp.log1p(jnp.exp(-jnp.abs(z)))

    total = jnp.sum(softplus(-pos_ref[...])) + jnp.sum(softplus(neg_ref[...]))
    out_ref[0, 0] = total / B


def kernel(emb, w_relation, src_pos, dst_pos, edge_type, src_neg, dst_neg):
    i32 = jnp.int32
    embP = _pack(emb.T)
    w_pad = jnp.pad(w_relation, ((0, 0), (0, H)))
    pos_score, neg_score = _sc_score(
        embP, w_pad, src_pos.astype(i32), dst_pos.astype(i32),
        edge_type.astype(i32), src_neg.astype(i32), dst_neg.astype(i32))

    loss = pl.pallas_call(
        _loss_body,
        out_shape=jax.ShapeDtypeStruct((1, 1), jnp.float32),
        out_specs=pl.BlockSpec(memory_space=pltpu.SMEM),
    )(pos_score.reshape(B // 128, 128), neg_score.reshape(BN // 128, 128))
    return loss[0, 0]


# speedup vs baseline: 1.0030x; 1.0030x over previous
import functools

import jax
import jax.numpy as jnp
from jax import lax
from jax.experimental import pallas as pl
from jax.experimental.pallas import tpu as pltpu
from jax.experimental.pallas import tpu_sc as plsc

N_NODES = 1000000
H = 64
NUM_RELS = 1000
B = 16384
NEG = 20
BN = B * NEG

NC = 2
NS = 16
L = 16
NW = NC * NS

POS_PER_W = B // NW
NEG_PER_W = BN // NW
CHUNK = 128
NCHUNK = NEG_PER_W // CHUNK
NPAIR = NCHUNK // 2

NB_TC = 16384
GRID_TC = (N_NODES + NB_TC - 1) // NB_TC
HALF_TC = NB_TC // 2
WIDE_ROWS = GRID_TC * HALF_TC


def _pack_body(in_ref, out_ref):
    blk = in_ref[...]
    eye = jnp.eye(H, dtype=jnp.float32)
    t = lax.dot_general(blk, eye, (((0,), (0,)), ((), ())),
                        preferred_element_type=jnp.float32)
    out_ref[:, pl.ds(0, H)] = t[0:HALF_TC]
    out_ref[:, pl.ds(H, H)] = t[HALF_TC:NB_TC]


def _pack(embT):
    return pl.pallas_call(
        _pack_body,
        grid=(GRID_TC,),
        in_specs=[pl.BlockSpec((H, NB_TC), lambda c: (0, c))],
        out_specs=pl.BlockSpec((HALF_TC, 2 * H), lambda c: (c, 0)),
        out_shape=jax.ShapeDtypeStruct((WIDE_ROWS, 2 * H), jnp.float32),
    )(embT)


def _xform(src, off, gidx, par):
    for v in range(CHUNK // L):
        g = src[pl.ds(off + L * v, L)]
        par[pl.ds(L * v, L)] = (g >> 13) & 1
        gidx[pl.ds(L * v, L)] = ((g >> 14) << 13) | (g & 8191)


def _score_rows(heads, tails, parh, part, w_for_row, scores, psum, t,
                score_off):
    lane = jnp.arange(L, dtype=jnp.int32)
    rbase = t * 16
    pvh = parh[pl.ds(rbase, L)]
    pvt = part[pl.ds(rbase, L)]
    for p in range(16):
        rl = rbase + p
        bh = H * pvh[p]
        bt = H * pvt[p]
        acc = jnp.zeros((L,), jnp.float32)
        for k in range(H // L):
            hk = heads[rl, pl.ds(bh + 16 * k, 16)]
            tk = tails[rl, pl.ds(bt + 16 * k, 16)]
            wk = w_for_row(rl, k, lane)
            acc = acc + hk * wk * tk
        psum[p, pl.ds(0, L)] = acc
    svec = jnp.zeros((L,), jnp.float32)
    for j in range(L):
        svec = svec + plsc.load_gather(
            psum, [lane, jnp.full((L,), j, jnp.int32)])
    scores[pl.ds(score_off + rbase, 16)] = svec
    return 0


def _sc_body(emb, w_pad, src_pos, dst_pos, etype, src_neg, dst_neg,
             pos_out, neg_out,
             pidx, didx, eidx, hidx0, hidx1, tidx0, tidx1,
             parh0, parh1, part0, part1,
             wrows, heads0, heads1, tails0, tails1,
             pos_sc, neg_sc, psum,
             sem_w, sem_g0, sem_g1, sem_i0, sem_i1):
    wid = lax.axis_index("s") * NC + lax.axis_index("c")
    pos0 = wid * POS_PER_W
    neg_base = wid * NEG_PER_W

    pltpu.sync_copy(etype.at[pl.ds(pos0, POS_PER_W)], eidx)
    pltpu.sync_copy(src_pos.at[pl.ds(pos0, POS_PER_W)], pidx)
    pltpu.sync_copy(dst_pos.at[pl.ds(pos0, POS_PER_W)], didx)
    for r in range(POS_PER_W // CHUNK):
        pltpu.async_copy(w_pad.at[eidx.at[pl.ds(r * CHUNK, CHUNK)]],
                         heads0, sem_w)
        pltpu.make_async_copy(w_pad.at[eidx.at[pl.ds(r * CHUNK, CHUNK)]],
                              heads0, sem_w).wait()

        def wcomp(i, carry, _r=r):
            b = _r * CHUNK + i
            for k in range(H // L):
                wrows[b >> 1, pl.ds(H * (b & 1) + 16 * k, 16)] = (
                    heads0[i, pl.ds(16 * k, 16)])
            return carry

        lax.fori_loop(0, CHUNK, wcomp, 0)

    for r in range(POS_PER_W // CHUNK):
        _xform(pidx, r * CHUNK, hidx0, parh0)
        _xform(didx, r * CHUNK, tidx0, part0)
        pltpu.async_copy(emb.at[hidx0], heads0, sem_g0)
        pltpu.async_copy(emb.at[tidx0], tails0, sem_g0)
        pltpu.make_async_copy(emb.at[hidx0], heads0, sem_g0).wait()
        pltpu.make_async_copy(emb.at[tidx0], tails0, sem_g0).wait()

        def w_pos(rl, k, lane, _r=r):
            b = _r * CHUNK + rl
            f = H * b + 16 * k + lane
            return plsc.load_gather(wrows, [f >> 7, f & 127])

        def pos_tile(t, carry, _r=r):
            return _score_rows(heads0, tails0, parh0, part0, w_pos,
                               pos_sc, psum, t, _r * CHUNK)

        lax.fori_loop(0, CHUNK // 16, pos_tile, 0)
    pltpu.sync_copy(pos_sc, pos_out.at[pl.ds(pos0, POS_PER_W)])

    def fire_idx(c, hb, tb, sem):
        off = neg_base + c * CHUNK
        pltpu.async_copy(src_neg.at[pl.ds(off, CHUNK)], hb, sem)
        pltpu.async_copy(dst_neg.at[pl.ds(off, CHUNK)], tb, sem)

    def wait_idx(hb, tb, sem):
        pltpu.make_async_copy(src_neg.at[pl.ds(0, CHUNK)], hb, sem).wait()
        pltpu.make_async_copy(dst_neg.at[pl.ds(0, CHUNK)], tb, sem).wait()

    def xform_fire(hb, tb, ph, pt, hd, td, sem):
        _xform(hb, 0, hb, ph)
        _xform(tb, 0, tb, pt)
        pltpu.async_copy(emb.at[hb], hd, sem)
        pltpu.async_copy(emb.at[tb], td, sem)

    def wait_g(hb, tb, hd, td, sem):
        pltpu.make_async_copy(emb.at[hb], hd, sem).wait()
        pltpu.make_async_copy(emb.at[tb], td, sem).wait()

    def compute(c, hd, td, ph, pt):
        def w_neg(rl, k, lane):
            fvec = lax.div(64 * (c * CHUNK + rl) + 16 * k + lane,
                           jnp.int32(20))
            return plsc.load_gather(wrows, [fvec >> 7, fvec & 127])

        def neg_tile(t, carry):
            return _score_rows(hd, td, ph, pt, w_neg, neg_sc, psum, t,
                               c * CHUNK)

        lax.fori_loop(0, CHUNK // 16, neg_tile, 0)

    fire_idx(0, hidx0, tidx0, sem_i0)
    fire_idx(1, hidx1, tidx1, sem_i1)
    wait_idx(hidx0, tidx0, sem_i0)
    xform_fire(hidx0, tidx0, parh0, part0, heads0, tails0, sem_g0)
    wait_idx(hidx1, tidx1, sem_i1)

    def pair(i, carry):
        c0 = 2 * i
        c1 = c0 + 1
        not_last = i < NPAIR - 1
        xform_fire(hidx1, tidx1, parh1, part1, heads1, tails1, sem_g1)
        wait_g(hidx0, tidx0, heads0, tails0, sem_g0)

        @pl.when(not_last)
        def _():
            fire_idx(c0 + 2, hidx0, tidx0, sem_i0)

        compute(c0, heads0, tails0, parh0, part0)

        @pl.when(not_last)
        def _():
            wait_idx(hidx0, tidx0, sem_i0)
            xform_fire(hidx0, tidx0, parh0, part0, heads0, tails0, sem_g0)

        wait_g(hidx1, tidx1, heads1, tails1, sem_g1)

        @pl.when(not_last)
        def _():
            fire_idx(c1 + 2, hidx1, tidx1, sem_i1)

        compute(c1, heads1, tails1, parh1, part1)

        @pl.when(not_last)
        def _():
            wait_idx(hidx1, tidx1, sem_i1)

        return carry

    lax.fori_loop(0, NPAIR, pair, 0)
    pltpu.sync_copy(neg_sc, neg_out.at[pl.ds(neg_base, NEG_PER_W)])


_sc_score = functools.partial(
    pl.kernel,
    out_type=(jax.ShapeDtypeStruct((B,), jnp.float32),
              jax.ShapeDtypeStruct((BN,), jnp.float32)),
    mesh=plsc.VectorSubcoreMesh(core_axis_name="c", subcore_axis_name="s"),
    compiler_params=pltpu.CompilerParams(needs_layout_passes=False,
                                         use_tc_tiling_on_sc=True),
    scratch_types=(
        pltpu.VMEM((POS_PER_W,), jnp.int32),
        pltpu.VMEM((POS_PER_W,), jnp.int32),
        pltpu.VMEM((POS_PER_W,), jnp.int32),
        pltpu.VMEM((CHUNK,), jnp.int32),
        pltpu.VMEM((CHUNK,), jnp.int32),
        pltpu.VMEM((CHUNK,), jnp.int32),
        pltpu.VMEM((CHUNK,), jnp.int32),
        pltpu.VMEM((CHUNK,), jnp.int32),
        pltpu.VMEM((CHUNK,), jnp.int32),
        pltpu.VMEM((CHUNK,), jnp.int32),
        pltpu.VMEM((CHUNK,), jnp.int32),
        pltpu.VMEM((POS_PER_W // 2, 2 * H), jnp.float32),
        pltpu.VMEM((CHUNK, 2 * H), jnp.float32),
        pltpu.VMEM((CHUNK, 2 * H), jnp.float32),
        pltpu.VMEM((CHUNK, 2 * H), jnp.float32),
        pltpu.VMEM((CHUNK, 2 * H), jnp.float32),
        pltpu.VMEM((POS_PER_W,), jnp.float32),
        pltpu.VMEM((NEG_PER_W,), jnp.float32),
        pltpu.VMEM((L, 128), jnp.float32),
        pltpu.SemaphoreType.DMA,
        pltpu.SemaphoreType.DMA,
        pltpu.SemaphoreType.DMA,
        pltpu.SemaphoreType.DMA,
        pltpu.SemaphoreType.DMA,
    ),
)(_sc_body)


def _loss_body(pos_ref, neg_ref, out_ref):
    def softplus(z):
        return jnp.maximum(z, 0.0) + jnp.log1p(jnp.exp(-jnp.abs(z)))

    total = jnp.sum(softplus(-pos_ref[...])) + jnp.sum(softplus(neg_ref[...]))
    out_ref[0, 0] = total / B


def kernel(emb, w_relation, src_pos, dst_pos, edge_type, src_neg, dst_neg):
    i32 = jnp.int32
    embP = _pack(emb.T)
    w_pad = jnp.pad(w_relation, ((0, 0), (0, H)))
    pos_score, neg_score = _sc_score(
        embP, w_pad, src_pos.astype(i32), dst_pos.astype(i32),
        edge_type.astype(i32), src_neg.astype(i32), dst_neg.astype(i32))

    loss = pl.pallas_call(
        _loss_body,
        out_shape=jax.ShapeDtypeStruct((1, 1), jnp.float32),
        out_specs=pl.BlockSpec(memory_space=pltpu.SMEM),
    )(pos_score.reshape(B // 128, 128), neg_score.reshape(BN // 128, 128))
    return loss[0, 0]


# speedup vs baseline: 1.0331x; 1.0301x over previous
import functools

import jax
import jax.numpy as jnp
from jax import lax
from jax.experimental import pallas as pl
from jax.experimental.pallas import tpu as pltpu
from jax.experimental.pallas import tpu_sc as plsc

N_NODES = 1000000
H = 64
NUM_RELS = 1000
B = 16384
NEG = 20
BN = B * NEG

NC = 2
NS = 16
L = 16
NW = NC * NS

POS_PER_W = B // NW
NEG_PER_W = BN // NW
CHUNK = 128
NCHUNK = NEG_PER_W // CHUNK
NPAIR = NCHUNK // 2

NB_TC = 32768
GRID_TC = (N_NODES + NB_TC - 1) // NB_TC
HALF_TC = NB_TC // 2
WIDE_ROWS = GRID_TC * HALF_TC


def _pack_body(in_ref, out_ref):
    blk = in_ref[...]
    eye = jnp.eye(H, dtype=jnp.float32)
    t = lax.dot_general(blk, eye, (((0,), (0,)), ((), ())),
                        preferred_element_type=jnp.float32)
    out_ref[:, pl.ds(0, H)] = t[0:HALF_TC]
    out_ref[:, pl.ds(H, H)] = t[HALF_TC:NB_TC]


def _pack(embT):
    return pl.pallas_call(
        _pack_body,
        grid=(GRID_TC,),
        in_specs=[pl.BlockSpec((H, NB_TC), lambda c: (0, c))],
        out_specs=pl.BlockSpec((HALF_TC, 2 * H), lambda c: (c, 0)),
        out_shape=jax.ShapeDtypeStruct((WIDE_ROWS, 2 * H), jnp.float32),
    )(embT)


def _xform(src, off, gidx, par):
    for v in range(CHUNK // L):
        g = src[pl.ds(off + L * v, L)]
        par[pl.ds(L * v, L)] = (g >> 14) & 1
        gidx[pl.ds(L * v, L)] = ((g >> 15) << 14) | (g & 16383)


def _score_rows(heads, tails, parh, part, w_for_row, scores, psum, t,
                score_off):
    lane = jnp.arange(L, dtype=jnp.int32)
    rbase = t * 16
    pvh = parh[pl.ds(rbase, L)]
    pvt = part[pl.ds(rbase, L)]
    for p in range(16):
        rl = rbase + p
        bh = H * pvh[p]
        bt = H * pvt[p]
        acc = jnp.zeros((L,), jnp.float32)
        for k in range(H // L):
            hk = heads[rl, pl.ds(bh + 16 * k, 16)]
            tk = tails[rl, pl.ds(bt + 16 * k, 16)]
            wk = w_for_row(rl, k, lane)
            acc = acc + hk * wk * tk
        psum[p, pl.ds(0, L)] = acc
    svec = jnp.zeros((L,), jnp.float32)
    for j in range(L):
        svec = svec + plsc.load_gather(
            psum, [lane, jnp.full((L,), j, jnp.int32)])
    scores[pl.ds(score_off + rbase, 16)] = svec
    return 0


def _sc_body(emb, w_pad, src_pos, dst_pos, etype, src_neg, dst_neg,
             pos_out, neg_out,
             pidx, didx, eidx, hidx0, hidx1, tidx0, tidx1,
             parh0, parh1, part0, part1,
             wrows, heads0, heads1, tails0, tails1,
             pos_sc, neg_sc, psum,
             sem_w, sem_g0, sem_g1, sem_i0, sem_i1):
    wid = lax.axis_index("s") * NC + lax.axis_index("c")
    pos0 = wid * POS_PER_W
    neg_base = wid * NEG_PER_W

    pltpu.sync_copy(etype.at[pl.ds(pos0, POS_PER_W)], eidx)
    pltpu.sync_copy(src_pos.at[pl.ds(pos0, POS_PER_W)], pidx)
    pltpu.sync_copy(dst_pos.at[pl.ds(pos0, POS_PER_W)], didx)
    for r in range(POS_PER_W // CHUNK):
        pltpu.async_copy(w_pad.at[eidx.at[pl.ds(r * CHUNK, CHUNK)]],
                         heads0, sem_w)
        pltpu.make_async_copy(w_pad.at[eidx.at[pl.ds(r * CHUNK, CHUNK)]],
                              heads0, sem_w).wait()

        def wcomp(i, carry, _r=r):
            b = _r * CHUNK + i
            for k in range(H // L):
                wrows[b >> 1, pl.ds(H * (b & 1) + 16 * k, 16)] = (
                    heads0[i, pl.ds(16 * k, 16)])
            return carry

        lax.fori_loop(0, CHUNK, wcomp, 0)

    for r in range(POS_PER_W // CHUNK):
        _xform(pidx, r * CHUNK, hidx0, parh0)
        _xform(didx, r * CHUNK, tidx0, part0)
        pltpu.async_copy(emb.at[hidx0], heads0, sem_g0)
        pltpu.async_copy(emb.at[tidx0], tails0, sem_g0)
        pltpu.make_async_copy(emb.at[hidx0], heads0, sem_g0).wait()
        pltpu.make_async_copy(emb.at[tidx0], tails0, sem_g0).wait()

        def w_pos(rl, k, lane, _r=r):
            b = _r * CHUNK + rl
            f = H * b + 16 * k + lane
            return plsc.load_gather(wrows, [f >> 7, f & 127])

        def pos_tile(t, carry, _r=r):
            return _score_rows(heads0, tails0, parh0, part0, w_pos,
                               pos_sc, psum, t, _r * CHUNK)

        lax.fori_loop(0, CHUNK // 16, pos_tile, 0)
    pltpu.sync_copy(pos_sc, pos_out.at[pl.ds(pos0, POS_PER_W)])

    def fire_idx(c, hb, tb, sem):
        off = neg_base + c * CHUNK
        pltpu.async_copy(src_neg.at[pl.ds(off, CHUNK)], hb, sem)
        pltpu.async_copy(dst_neg.at[pl.ds(off, CHUNK)], tb, sem)

    def wait_idx(hb, tb, sem):
        pltpu.make_async_copy(src_neg.at[pl.ds(0, CHUNK)], hb, sem).wait()
        pltpu.make_async_copy(dst_neg.at[pl.ds(0, CHUNK)], tb, sem).wait()

    def xform_fire(hb, tb, ph, pt, hd, td, sem):
        _xform(hb, 0, hb, ph)
        _xform(tb, 0, tb, pt)
        pltpu.async_copy(emb.at[hb], hd, sem)
        pltpu.async_copy(emb.at[tb], td, sem)

    def wait_g(hb, tb, hd, td, sem):
        pltpu.make_async_copy(emb.at[hb], hd, sem).wait()
        pltpu.make_async_copy(emb.at[tb], td, sem).wait()

    def compute(c, hd, td, ph, pt):
        def w_neg(rl, k, lane):
            fvec = lax.div(64 * (c * CHUNK + rl) + 16 * k + lane,
                           jnp.int32(20))
            return plsc.load_gather(wrows, [fvec >> 7, fvec & 127])

        def neg_tile(t, carry):
            return _score_rows(hd, td, ph, pt, w_neg, neg_sc, psum, t,
                               c * CHUNK)

        lax.fori_loop(0, CHUNK // 16, neg_tile, 0)

    fire_idx(0, hidx0, tidx0, sem_i0)
    fire_idx(1, hidx1, tidx1, sem_i1)
    wait_idx(hidx0, tidx0, sem_i0)
    xform_fire(hidx0, tidx0, parh0, part0, heads0, tails0, sem_g0)
    wait_idx(hidx1, tidx1, sem_i1)

    def pair(i, carry):
        c0 = 2 * i
        c1 = c0 + 1
        not_last = i < NPAIR - 1
        xform_fire(hidx1, tidx1, parh1, part1, heads1, tails1, sem_g1)
        wait_g(hidx0, tidx0, heads0, tails0, sem_g0)

        @pl.when(not_last)
        def _():
            fire_idx(c0 + 2, hidx0, tidx0, sem_i0)

        compute(c0, heads0, tails0, parh0, part0)

        @pl.when(not_last)
        def _():
            wait_idx(hidx0, tidx0, sem_i0)
            xform_fire(hidx0, tidx0, parh0, part0, heads0, tails0, sem_g0)

        wait_g(hidx1, tidx1, heads1, tails1, sem_g1)

        @pl.when(not_last)
        def _():
            fire_idx(c1 + 2, hidx1, tidx1, sem_i1)

        compute(c1, heads1, tails1, parh1, part1)

        @pl.when(not_last)
        def _():
            wait_idx(hidx1, tidx1, sem_i1)

        return carry

    lax.fori_loop(0, NPAIR, pair, 0)
    pltpu.sync_copy(neg_sc, neg_out.at[pl.ds(neg_base, NEG_PER_W)])


_sc_score = functools.partial(
    pl.kernel,
    out_type=(jax.ShapeDtypeStruct((B,), jnp.float32),
              jax.ShapeDtypeStruct((BN,), jnp.float32)),
    mesh=plsc.VectorSubcoreMesh(core_axis_name="c", subcore_axis_name="s"),
    compiler_params=pltpu.CompilerParams(needs_layout_passes=False,
                                         use_tc_tiling_on_sc=True),
    scratch_types=(
        pltpu.VMEM((POS_PER_W,), jnp.int32),
        pltpu.VMEM((POS_PER_W,), jnp.int32),
        pltpu.VMEM((POS_PER_W,), jnp.int32),
        pltpu.VMEM((CHUNK,), jnp.int32),
        pltpu.VMEM((CHUNK,), jnp.int32),
        pltpu.VMEM((CHUNK,), jnp.int32),
        pltpu.VMEM((CHUNK,), jnp.int32),
        pltpu.VMEM((CHUNK,), jnp.int32),
        pltpu.VMEM((CHUNK,), jnp.int32),
        pltpu.VMEM((CHUNK,), jnp.int32),
        pltpu.VMEM((CHUNK,), jnp.int32),
        pltpu.VMEM((POS_PER_W // 2, 2 * H), jnp.float32),
        pltpu.VMEM((CHUNK, 2 * H), jnp.float32),
        pltpu.VMEM((CHUNK, 2 * H), jnp.float32),
        pltpu.VMEM((CHUNK, 2 * H), jnp.float32),
        pltpu.VMEM((CHUNK, 2 * H), jnp.float32),
        pltpu.VMEM((POS_PER_W,), jnp.float32),
        pltpu.VMEM((NEG_PER_W,), jnp.float32),
        pltpu.VMEM((L, 128), jnp.float32),
        pltpu.SemaphoreType.DMA,
        pltpu.SemaphoreType.DMA,
        pltpu.SemaphoreType.DMA,
        pltpu.SemaphoreType.DMA,
        pltpu.SemaphoreType.DMA,
    ),
)(_sc_body)


def _loss_body(pos_ref, neg_ref, out_ref):
    def softplus(z):
        return jnp.maximum(z, 0.0) + jnp.log1p(jnp.exp(-jnp.abs(z)))

    total = jnp.sum(softplus(-pos_ref[...])) + jnp.sum(softplus(neg_ref[...]))
    out_ref[0, 0] = total / B


def kernel(emb, w_relation, src_pos, dst_pos, edge_type, src_neg, dst_neg):
    i32 = jnp.int32
    embP = _pack(emb.T)
    w_pad = jnp.pad(w_relation, ((0, 0), (0, H)))
    pos_score, neg_score = _sc_score(
        embP, w_pad, src_pos.astype(i32), dst_pos.astype(i32),
        edge_type.astype(i32), src_neg.astype(i32), dst_neg.astype(i32))

    loss = pl.pallas_call(
        _loss_body,
        out_shape=jax.ShapeDtypeStruct((1, 1), jnp.float32),
        out_specs=pl.BlockSpec(memory_space=pltpu.SMEM),
    )(pos_score.reshape(B // 128, 128), neg_score.reshape(BN // 128, 128))
    return loss[0, 0]
